# baseline (speedup 1.0000x reference)
import jax
import jax.numpy as jnp
from jax.experimental import pallas as pl
from jax.experimental.pallas import tpu as pltpu

_B_BLOCK = 512


def _conv_kernel(x_ref, w_ref, b_ref, o_ref):
    npairs = w_ref.shape[0]
    for p in range(npairs):
        lo, hi = p * 128, (p + 1) * 128
        o_ref[:, lo:hi] = (
            jnp.dot(x_ref[:, lo:hi], w_ref[p], preferred_element_type=jnp.float32)
            + b_ref[p][None, :]
        )


def kernel(x, W, bias):
    B, G, CIN = x.shape
    COUT = W.shape[2]
    npairs = G // 2

    Wr = W.reshape(npairs, 2, CIN, COUT)
    Wp = jnp.zeros((npairs, 2 * CIN, 2 * COUT), dtype=W.dtype)
    Wp = Wp.at[:, :CIN, :COUT].set(Wr[:, 0])
    Wp = Wp.at[:, CIN:, COUT:].set(Wr[:, 1])
    bp = bias.reshape(npairs, 2 * COUT)

    x2 = x.reshape(B, G * CIN)

    out2 = pl.pallas_call(
        _conv_kernel,
        grid=(B // _B_BLOCK,),
        in_specs=[
            pl.BlockSpec((_B_BLOCK, G * CIN), lambda i: (i, 0)),
            pl.BlockSpec((npairs, 2 * CIN, 2 * COUT), lambda i: (0, 0, 0)),
            pl.BlockSpec((npairs, 2 * COUT), lambda i: (0, 0)),
        ],
        out_specs=pl.BlockSpec((_B_BLOCK, G * COUT), lambda i: (i, 0)),
        out_shape=jax.ShapeDtypeStruct((B, G * COUT), x.dtype),
        compiler_params=pltpu.CompilerParams(
            dimension_semantics=("arbitrary",),
        ),
    )(x2, Wp, bp)

    return out2.reshape(B, G, COUT)


# speedup vs baseline: 1.0021x; 1.0021x over previous
import jax
import jax.numpy as jnp
from jax.experimental import pallas as pl
from jax.experimental.pallas import tpu as pltpu

_B_BLOCK = 1024


def _conv_kernel(x_ref, w_ref, b_ref, o_ref):
    npairs = w_ref.shape[0]
    for p in range(npairs):
        lo, hi = p * 128, (p + 1) * 128
        xb = x_ref[:, lo:hi].astype(jnp.bfloat16)
        o_ref[:, lo:hi] = (
            jnp.dot(xb, w_ref[p], preferred_element_type=jnp.float32)
            + b_ref[p][None, :]
        )


def kernel(x, W, bias):
    B, G, CIN = x.shape
    COUT = W.shape[2]
    npairs = G // 2

    Wr = W.reshape(npairs, 2, CIN, COUT)
    Wp = jnp.zeros((npairs, 2 * CIN, 2 * COUT), dtype=jnp.bfloat16)
    Wp = Wp.at[:, :CIN, :COUT].set(Wr[:, 0].astype(jnp.bfloat16))
    Wp = Wp.at[:, CIN:, COUT:].set(Wr[:, 1].astype(jnp.bfloat16))
    bp = bias.reshape(npairs, 2 * COUT)

    x2 = x.reshape(B, G * CIN)

    out2 = pl.pallas_call(
        _conv_kernel,
        grid=(B // _B_BLOCK,),
        in_specs=[
            pl.BlockSpec((_B_BLOCK, G * CIN), lambda i: (i, 0)),
            pl.BlockSpec((npairs, 2 * CIN, 2 * COUT), lambda i: (0, 0, 0)),
            pl.BlockSpec((npairs, 2 * COUT), lambda i: (0, 0)),
        ],
        out_specs=pl.BlockSpec((_B_BLOCK, G * COUT), lambda i: (i, 0)),
        out_shape=jax.ShapeDtypeStruct((B, G * COUT), x.dtype),
        compiler_params=pltpu.CompilerParams(
            dimension_semantics=("parallel",),
        ),
    )(x2, Wp, bp)

    return out2.reshape(B, G, COUT)


# speedup vs baseline: 2.2988x; 2.2940x over previous
import jax
import jax.numpy as jnp
from jax.experimental import pallas as pl
from jax.experimental.pallas import tpu as pltpu

_B_BLOCK = 4096


def _conv_kernel(x_ref, w_ref, b_ref, o_ref):
    xg = x_ref[0].astype(jnp.bfloat16)
    o_ref[0] = (
        jnp.dot(w_ref[0], xg, preferred_element_type=jnp.float32)
        + b_ref[0, 0][:, None]
    )


def kernel(x, W, bias):
    B, G, CIN = x.shape
    COUT = W.shape[2]

    xT = jnp.transpose(x, (1, 2, 0))
    WT = jnp.transpose(W, (0, 2, 1)).astype(jnp.bfloat16)

    outT = pl.pallas_call(
        _conv_kernel,
        grid=(G, B // _B_BLOCK),
        in_specs=[
            pl.BlockSpec((1, CIN, _B_BLOCK), lambda g, i: (g, 0, i)),
            pl.BlockSpec((1, COUT, CIN), lambda g, i: (g, 0, 0)),
            pl.BlockSpec((1, 1, COUT), lambda g, i: (g, 0, 0)),
        ],
        out_specs=pl.BlockSpec((1, COUT, _B_BLOCK), lambda g, i: (g, 0, i)),
        out_shape=jax.ShapeDtypeStruct((G, COUT, B), x.dtype),
        compiler_params=pltpu.CompilerParams(
            dimension_semantics=("parallel", "parallel"),
        ),
    )(xT, WT, bias.reshape(G, 1, COUT))

    return jnp.transpose(outT, (2, 0, 1))


# speedup vs baseline: 3.8296x; 1.6659x over previous
import jax
import jax.numpy as jnp
from jax.experimental import pallas as pl
from jax.experimental.pallas import tpu as pltpu

_B_BLOCK = 4096


def _conv_kernel(x_ref, w_ref, b_ref, o_ref):
    xg = x_ref[0].astype(jnp.bfloat16)
    o_ref[0] = (
        jnp.dot(w_ref[0], xg, preferred_element_type=jnp.float32)
        + b_ref[0, 0][:, None]
    )


def kernel(x, W, bias):
    B, G, CIN = x.shape
    COUT = W.shape[2]

    xT = jnp.transpose(x, (1, 2, 0))
    WT = jnp.transpose(W, (0, 2, 1)).astype(jnp.bfloat16)

    outT = pl.pallas_call(
        _conv_kernel,
        grid=(G,),
        in_specs=[
            pl.BlockSpec((1, CIN, B), lambda g: (g, 0, 0)),
            pl.BlockSpec((1, COUT, CIN), lambda g: (g, 0, 0)),
            pl.BlockSpec((1, 1, COUT), lambda g: (g, 0, 0)),
        ],
        out_specs=pl.BlockSpec((1, COUT, B), lambda g: (g, 0, 0)),
        out_shape=jax.ShapeDtypeStruct((G, COUT, B), x.dtype),
        compiler_params=pltpu.CompilerParams(
            dimension_semantics=("parallel",),
        ),
    )(xT, WT, bias.reshape(G, 1, COUT))

    return jnp.transpose(outT, (2, 0, 1))
